# baseline (speedup 1.0000x reference)
import functools
import math
import jax
import jax.numpy as jnp
from jax import lax
from jax.experimental import pallas as pl
from jax.experimental.pallas import tpu as pltpu
from jax.experimental.pallas import tpu_sc as plsc

_M = 4
_G = 512
_HW = 1024
_NB = 32
_BLK = _G * _HW
_CH = 32
_NCHUNK = _G // _CH
_CHW = _CH * _HW
_STRIPS = _HW // 16
_INV_SCALE = 1.0 / math.sqrt(_G)
_NEG = -3.0e38

_mesh = plsc.VectorSubcoreMesh(
    core_axis_name="c", subcore_axis_name="s", num_cores=2, num_subcores=16
)


def _sc_body(x_hbm, q_hbm, buf_a, buf_b, mxv, ixv, si0, si1, so0, so1):
    w = lax.axis_index("s") * 2 + lax.axis_index("c")
    bufs = (buf_a, buf_b)
    sin = (si0, si1)
    sout = (so0, so1)
    lane = lax.iota(jnp.int32, 16)
    negv = jnp.full((16,), _NEG, jnp.float32)
    zi = jnp.zeros((16,), jnp.int32)
    onesv = jnp.full((16,), 1.0, jnp.float32)
    zerov = jnp.zeros((16,), jnp.float32)

    def init_body(j, c):
        off = pl.multiple_of(j * 16, 16)
        mxv[pl.ds(off, 16)] = negv
        ixv[pl.ds(off, 16)] = zi
        return c

    lax.fori_loop(0, _STRIPS, init_body, 0)

    base = w * _BLK
    handles = [pltpu.async_copy(x_hbm.at[pl.ds(base, _CHW)], buf_a, si0), None]
    for k in range(_NCHUNK):
        b = k & 1
        handles[b].wait()
        if k + 1 < _NCHUNK:
            nxt = (k + 1) & 1
            handles[nxt] = pltpu.async_copy(
                x_hbm.at[pl.ds(base + (k + 1) * _CHW, _CHW)], bufs[nxt], sin[nxt]
            )
        buf = bufs[b]

        def strip_body(j, c, buf=buf, k=k):
            off = pl.multiple_of(j * 16, 16)
            m = mxv[pl.ds(off, 16)]
            i = ixv[pl.ds(off, 16)]
            for r in range(_CH):
                v = buf[pl.ds(off + r * _HW, 16)]
                gt = v > m
                m = jnp.where(gt, v, m)
                i = jnp.where(gt, jnp.full((16,), k * _CH + r, jnp.int32), i)
            mxv[pl.ds(off, 16)] = m
            ixv[pl.ds(off, 16)] = i
            return c

        lax.fori_loop(0, _STRIPS, strip_body, 0)

    def zero_body(t, c, buf=None):
        off = pl.multiple_of(t * 64, 16)
        for u in range(4):
            buf[pl.ds(off + u * 16, 16)] = zerov
        return c

    lax.fori_loop(0, _CHW // 64, functools.partial(zero_body, buf=buf_a), 0)
    lax.fori_loop(0, _CHW // 64, functools.partial(zero_body, buf=buf_b), 0)

    def scat(buf, k, val):
        def body(j, c):
            off = pl.multiple_of(j * 16, 16)
            iv = ixv[pl.ds(off, 16)]
            local = iv - (k * _CH)
            msk = (local >= 0) & (local < _CH)
            addr = local * _HW + off + lane
            addr = jnp.where(msk, addr, zi)
            plsc.store_scatter(buf, [addr], val, mask=msk)
            return c

        lax.fori_loop(0, _STRIPS, body, 0)

    out_h = [None, None]
    for k in range(_NCHUNK):
        b = k & 1
        if out_h[b] is not None:
            out_h[b].wait()
            scat(bufs[b], k - 2, zerov)
        scat(bufs[b], k, onesv)
        out_h[b] = pltpu.async_copy(
            bufs[b], q_hbm.at[pl.ds(base + k * _CHW, _CHW)], sout[b]
        )
    out_h[0].wait()
    out_h[1].wait()


_sc_quantize = functools.partial(
    pl.kernel,
    out_type=jax.ShapeDtypeStruct((_NB * _BLK,), jnp.float32),
    mesh=_mesh,
    compiler_params=pltpu.CompilerParams(needs_layout_passes=False),
    scratch_types=[
        pltpu.VMEM((_CHW,), jnp.float32),
        pltpu.VMEM((_CHW,), jnp.float32),
        pltpu.VMEM((_HW,), jnp.float32),
        pltpu.VMEM((_HW,), jnp.int32),
        pltpu.SemaphoreType.DMA,
        pltpu.SemaphoreType.DMA,
        pltpu.SemaphoreType.DMA,
        pltpu.SemaphoreType.DMA,
    ],
)(_sc_body)


def _tc_body(x_ref, l_ref):
    l_ref[...] = x_ref[...] * _INV_SCALE


def _tc_scale(x128):
    blk = (8192, 128)
    return pl.pallas_call(
        _tc_body,
        grid=(x128.shape[0] // blk[0],),
        in_specs=[pl.BlockSpec(blk, lambda i: (i, 0))],
        out_specs=pl.BlockSpec(blk, lambda i: (i, 0)),
        out_shape=jax.ShapeDtypeStruct(x128.shape, x128.dtype),
    )(x128)


def kernel(logit, temperature):
    n, c, h, w = logit.shape
    g = c // _M
    x1 = logit.reshape(_NB * _BLK)
    q1 = _sc_quantize(x1)
    l2 = _tc_scale(logit.reshape(_NB * _BLK // 128, 128))
    return q1.reshape(n, c, h, w), l2.reshape(n, _M, g, h, w)


# speedup vs baseline: 3.4799x; 3.4799x over previous
import functools
import math
import jax
import jax.numpy as jnp
from jax import lax
from jax.experimental import pallas as pl
from jax.experimental.pallas import tpu as pltpu
from jax.experimental.pallas import tpu_sc as plsc

_M = 4
_G = 512
_NG = 32768
_GPT = 1024
_BLK = _GPT * _G
_CH = 64
_NCHUNK = _GPT // _CH
_CHW = _CH * _G
_NSET = _CH // 16
_INV_SCALE = 1.0 / math.sqrt(_G)
_NEG = -3.0e38

_mesh = plsc.VectorSubcoreMesh(
    core_axis_name="c", subcore_axis_name="s", num_cores=2, num_subcores=16
)


def _sc_body(x_hbm, q_hbm, buf_a, buf_b, ixv, si0, si1, so0, so1):
    w = lax.axis_index("s") * 2 + lax.axis_index("c")
    base = w * _BLK
    bufs = (buf_a, buf_b)
    sin = (si0, si1)
    sout = (so0, so1)
    lane = lax.iota(jnp.int32, 16)
    negv = jnp.full((16,), _NEG, jnp.float32)
    onesv = jnp.full((16,), 1.0, jnp.float32)
    zerov = jnp.zeros((16,), jnp.float32)

    def start_in(b, k):
        off = pl.multiple_of(base + k * _CHW, 256)
        return pltpu.async_copy(x_hbm.at[pl.ds(off, _CHW)], bufs[b], sin[b])

    def wait_in(b):
        pltpu.make_async_copy(x_hbm.at[pl.ds(0, _CHW)], bufs[b], sin[b]).wait()

    def start_out(b, k):
        off = pl.multiple_of(base + k * _CHW, 256)
        return pltpu.async_copy(bufs[b], q_hbm.at[pl.ds(off, _CHW)], sout[b])

    def wait_out(b):
        pltpu.make_async_copy(bufs[b], q_hbm.at[pl.ds(0, _CHW)], sout[b]).wait()

    set_bases = []
    for s in range(_NSET):
        u, hh = s // 2, s % 2
        set_bases.append(
            (u * 16384 + hh * 2 * 4096)
            + (lane // 8) * 4096
            + (lane % 8) * 128
        )

    def compute_chunk(b, k):
        buf = bufs[b]
        for s in range(_NSET):
            av = set_bases[s]

            def step(t, carry, buf=buf):
                m, ri, addrv = carry
                for _ in range(31):
                    v = plsc.load_gather(buf, [addrv])
                    gt = v > m
                    m = jnp.where(gt, v, m)
                    ri = jnp.where(gt, addrv, ri)
                    addrv = addrv + 1
                v = plsc.load_gather(buf, [addrv])
                gt = v > m
                m = jnp.where(gt, v, m)
                ri = jnp.where(gt, addrv, ri)
                addrv = addrv + lax.select(t % 4 == 3, 897, 1)
                return m, ri, addrv

            m, ri, _ = lax.fori_loop(0, 16, step, (negv, av, av))
            diff = ri - av
            g = ((diff >> 10) << 7) | (diff & 127)
            ixv[pl.ds(pl.multiple_of(k * _CH + s * 16, 16), 16)] = g

    handles = [start_in(0, 0), start_in(1, 1)]
    del handles

    def p1_pair(kk, c):
        for b in range(2):
            k = kk * 2 + b
            wait_in(b)
            compute_chunk(b, k)
            start_in(b, k + 2)
        return c

    lax.fori_loop(0, (_NCHUNK - 2) // 2, p1_pair, 0)
    for b in range(2):
        wait_in(b)
        compute_chunk(b, _NCHUNK - 2 + b)

    def zero_buf(buf):
        def zb(t, c, buf=buf):
            off = pl.multiple_of(t * 64, 16)
            for u in range(4):
                buf[pl.ds(off + u * 16, 16)] = zerov
            return c

        lax.fori_loop(0, _CHW // 64, zb, 0)

    zero_buf(buf_a)
    zero_buf(buf_b)

    def scat(b, k, val):
        buf = bufs[b]
        for s in range(_NSET):
            iv = ixv[pl.ds(pl.multiple_of(k * _CH + s * 16, 16), 16)]
            addr = set_bases[s] + ((iv >> 7) << 10) + (iv & 127)
            plsc.store_scatter(buf, [addr], val)

    for b in range(2):
        scat(b, b, onesv)
        start_out(b, b)

    def p2_pair(kk, c):
        for b in range(2):
            k = kk * 2 + 2 + b
            wait_out(b)
            scat(b, k - 2, zerov)
            scat(b, k, onesv)
            start_out(b, k)
        return c

    lax.fori_loop(0, (_NCHUNK - 2) // 2, p2_pair, 0)
    wait_out(0)
    wait_out(1)


_sc_quantize = functools.partial(
    pl.kernel,
    out_type=jax.ShapeDtypeStruct((_NG * _G,), jnp.float32),
    mesh=_mesh,
    compiler_params=pltpu.CompilerParams(needs_layout_passes=False),
    scratch_types=[
        pltpu.VMEM((_CHW,), jnp.float32),
        pltpu.VMEM((_CHW,), jnp.float32),
        pltpu.VMEM((_GPT,), jnp.int32),
        pltpu.SemaphoreType.DMA,
        pltpu.SemaphoreType.DMA,
        pltpu.SemaphoreType.DMA,
        pltpu.SemaphoreType.DMA,
    ],
)(_sc_body)


def _tc_body(x_ref, l_ref):
    l_ref[...] = (x_ref[...] * _INV_SCALE).reshape(l_ref.shape)


def _tc_scale(xT):
    n, h, w, c = xT.shape
    in_blk = (1, h, w, _G)
    out_blk = (1, 1, h, w, _G)
    return pl.pallas_call(
        _tc_body,
        grid=(n, _M),
        in_specs=[pl.BlockSpec(in_blk, lambda i, m: (i, 0, 0, m))],
        out_specs=pl.BlockSpec(out_blk, lambda i, m: (i, m, 0, 0, 0)),
        out_shape=jax.ShapeDtypeStruct((n, _M, h, w, _G), xT.dtype),
    )(xT)


def kernel(logit, temperature):
    n, c, h, w = logit.shape
    g = c // _M
    xT = logit.transpose(0, 2, 3, 1)
    x_phys = (
        xT.reshape(n, h, w // 8, 8, c // 128, 128)
        .transpose(0, 1, 2, 4, 3, 5)
        .reshape(-1)
    )
    q1 = _sc_quantize(x_phys)
    lT = _tc_scale(xT)
    qT = (
        q1.reshape(n, h, w // 8, c // 128, 8, 128)
        .transpose(0, 1, 2, 4, 3, 5)
        .reshape(n, h, w, c)
    )
    q = qT.transpose(0, 3, 1, 2)
    l = lT.transpose(0, 1, 4, 2, 3)
    return q, l
